# baseline (speedup 1.0000x reference)
import functools

import jax
import jax.numpy as jnp
from jax import lax
from jax.experimental import pallas as pl
from jax.experimental.pallas import tpu as pltpu
from jax.experimental.pallas import tpu_sc as plsc

B = 16384
D = 64

_info = plsc.get_sparse_core_info()
_NC, _NS = _info.num_cores, _info.num_subcores
_NW = _NC * _NS
_BPW = B // _NW
_CHUNK = 128
_NCHUNK = _BPW // _CHUNK

_mesh = plsc.VectorSubcoreMesh(core_axis_name="c", subcore_axis_name="s")


@functools.partial(
    pl.kernel,
    mesh=_mesh,
    compiler_params=pltpu.CompilerParams(use_tc_tiling_on_sc=False),
    out_type=[jax.ShapeDtypeStruct((B, D), jnp.float32)] * 3,
    scratch_types=[
        pltpu.VMEM((_NCHUNK, _CHUNK), jnp.int32),
        pltpu.VMEM((_NCHUNK, _CHUNK), jnp.int32),
        pltpu.VMEM((_NCHUNK, _CHUNK), jnp.int32),
        pltpu.VMEM((_BPW, D), jnp.float32),
        pltpu.VMEM((_BPW, D), jnp.float32),
        pltpu.VMEM((_BPW, D), jnp.float32),
        pltpu.SemaphoreType.DMA,
        pltpu.SemaphoreType.DMA,
        pltpu.SemaphoreType.DMA,
    ],
)
def _sc_gather(idx0, idx1, idx2, t0, t1, t2, o0, o1, o2,
               iv0, iv1, iv2, r0, r1, r2, s0, s1, s2):
    wid = lax.axis_index("s") * _NC + lax.axis_index("c")
    base = wid * _BPW
    pltpu.sync_copy(idx0.at[wid], iv0)
    pltpu.sync_copy(idx1.at[wid], iv1)
    pltpu.sync_copy(idx2.at[wid], iv2)
    copies = []
    for j in range(_NCHUNK):
        dst = pl.ds(j * _CHUNK, _CHUNK)
        copies.append(pltpu.async_copy(t0.at[iv0.at[j]], r0.at[dst], s0))
        copies.append(pltpu.async_copy(t1.at[iv1.at[j]], r1.at[dst], s1))
        copies.append(pltpu.async_copy(t2.at[iv2.at[j]], r2.at[dst], s2))
    for c in copies:
        c.wait()
    pltpu.sync_copy(r0, o0.at[pl.ds(base, _BPW)])
    pltpu.sync_copy(r1, o1.at[pl.ds(base, _BPW)])
    pltpu.sync_copy(r2, o2.at[pl.ds(base, _BPW)])


_BM = 1024


def _mlp_body(e0, e1, e2, w1, b1, w2, b2, out):
    emb = jnp.concatenate([e0[...], e1[...], e2[...]], axis=1)
    h = jnp.dot(emb, w1[...], preferred_element_type=jnp.float32) + b1[...]
    h = jnp.maximum(h, 0.0)
    out[...] = jnp.dot(h, w2[...], preferred_element_type=jnp.float32) + b2[...]


def _tc_mlp(e0, e1, e2, W1, b1, W2, b2):
    n1, n2 = W1.shape[1], W2.shape[1]
    grid = (B // _BM,)
    return pl.pallas_call(
        _mlp_body,
        grid=grid,
        in_specs=[
            pl.BlockSpec((_BM, D), lambda i: (i, 0)),
            pl.BlockSpec((_BM, D), lambda i: (i, 0)),
            pl.BlockSpec((_BM, D), lambda i: (i, 0)),
            pl.BlockSpec((3 * D, n1), lambda i: (0, 0)),
            pl.BlockSpec((1, n1), lambda i: (0, 0)),
            pl.BlockSpec((n1, n2), lambda i: (0, 0)),
            pl.BlockSpec((1, n2), lambda i: (0, 0)),
        ],
        out_specs=pl.BlockSpec((_BM, n2), lambda i: (i, 0)),
        out_shape=jax.ShapeDtypeStruct((B, n2), jnp.float32),
    )(e0, e1, e2, W1, b1.reshape(1, n1), W2, b2.reshape(1, n2))


def kernel(x, table_item_id, table_category, table_brand, W1, b1, W2, b2):
    xi = x.astype(jnp.int32)
    idx0 = xi[:, 0].reshape(_NW, _NCHUNK, _CHUNK)
    idx1 = xi[:, 1].reshape(_NW, _NCHUNK, _CHUNK)
    idx2 = xi[:, 2].reshape(_NW, _NCHUNK, _CHUNK)
    e0, e1, e2 = _sc_gather(idx0, idx1, idx2,
                            table_item_id, table_category, table_brand)
    return _tc_mlp(e0, e1, e2, W1, b1, W2, b2)
